# baseline (speedup 1.0000x reference)
import jax
import jax.numpy as jnp
from jax import lax
from jax.experimental import pallas as pl
from jax.experimental.pallas import tpu as pltpu
from jax.experimental.pallas import tpu_sc as plsc

BATCH = 4096
HIST = 200
EMBED = 64
L = 16
HPAD = 208
HHALF = 104
NC, NS = 2, 16
NW = NC * NS
BPW = BATCH // NW
NSEG = 2 * BPW
KREG = EMBED // L
NIDX = 4
NROW = 2


def _rsqrt_newton(p):
    bits = plsc.bitcast(p, jnp.int32)
    bits = jnp.full((L,), 0x5F3759DF, jnp.int32) - (bits >> 1)
    y = plsc.bitcast(bits, jnp.float32)
    for _ in range(3):
        y = y * (1.5 - 0.5 * p * y * y)
    return y


def _lane_sum(red_v, x):
    lane = lax.iota(jnp.int32, L)
    for s in (8, 4, 2, 1):
        red_v[...] = x
        x = x + plsc.load_gather(red_v, [lane ^ s])
    return x


def _body(seq12_hbm, table_hbm, sim_hbm, vec1_hbm, vec2_hbm,
          idx_v, rows_v, vec_v, sim_v, red_v, sem_g, sem_i):
    wid = lax.axis_index("s") * NC + lax.axis_index("c")
    base = wid * BPW

    zf = jnp.zeros((L,), jnp.float32)
    for q in range(NIDX):
        idx_v[q, pl.ds(HPAD - L, L)] = jnp.zeros((L,), jnp.int32)

    def seg_off(s):
        off = jnp.where(s < BPW, (base + s) * HIST,
                        (BATCH + base + (s - BPW)) * HIST)
        return pl.multiple_of(off, 8)

    def idx_copy(s):
        pltpu.async_copy(seq12_hbm.at[pl.ds(seg_off(s), HIST)],
                         idx_v.at[s % NIDX, pl.ds(0, HIST)], sem_i)

    def idx_wait():
        pltpu.make_async_copy(seq12_hbm.at[pl.ds(0, HIST)],
                              idx_v.at[0, pl.ds(0, HIST)], sem_i).wait()

    def gather(s):
        for h in range(2):
            pltpu.async_copy(
                table_hbm.at[idx_v.at[s % NIDX, pl.ds(h * HHALF, HHALF)]],
                rows_v.at[s % NROW, pl.ds(h * HHALF, HHALF)], sem_g)

    def gather_wait():
        for h in range(2):
            pltpu.make_async_copy(table_hbm.at[pl.ds(0, HHALF)],
                                  rows_v.at[0, pl.ds(0, HHALF)], sem_g).wait()

    idx_copy(0)
    idx_wait()
    gather(0)
    idx_copy(1)

    def seg_body(s, carry):
        q, r = s % NIDX, s % NROW

        @pl.when(s + 1 < NSEG)
        def _():
            idx_wait()
            gather(s + 1)

        cnt = zf
        for j in range(HPAD // L):
            v = idx_v[q, pl.ds(j * L, L)]
            cnt = cnt + jnp.where(v != 0, 1.0, 0.0).astype(jnp.float32)

        @pl.when(s + 2 < NSEG)
        def _():
            idx_copy(s + 2)

        gather_wait()

        def red(j, acc):
            return tuple(acc[k] + rows_v[r, j, pl.ds(k * L, L)]
                         for k in range(KREG))

        acc = lax.fori_loop(0, HPAD, red, (zf,) * KREG, unroll=8)
        denom = jnp.maximum(_lane_sum(red_v, cnt), 1e-9)
        vs = tuple(acc[k] / denom for k in range(KREG))
        for k in range(KREG):
            vec_v[s, pl.ds(k * L, L)] = vs[k]

        @pl.when(s >= BPW)
        def _():
            row = s - BPW
            dot, n1, n2 = zf, zf, zf
            for k in range(KREG):
                v1k = vec_v[row, pl.ds(k * L, L)]
                dot = dot + v1k * vs[k]
                n1 = n1 + v1k * v1k
                n2 = n2 + vs[k] * vs[k]
            p = jnp.maximum(_lane_sum(red_v, n1) * _lane_sum(red_v, n2),
                            1e-16)
            sim = _lane_sum(red_v, dot) * _rsqrt_newton(p)
            lane = lax.iota(jnp.int32, L)
            plsc.store_scatter(sim_v, [jnp.full((L,), row, jnp.int32)],
                               sim, mask=lane == 0)

        return carry

    lax.fori_loop(0, NSEG, seg_body, 0)

    pltpu.sync_copy(sim_v, sim_hbm.at[pl.ds(base, BPW)])
    pltpu.sync_copy(vec_v.at[pl.ds(0, BPW)], vec1_hbm.at[pl.ds(base, BPW)])
    pltpu.sync_copy(vec_v.at[pl.ds(BPW, BPW)], vec2_hbm.at[pl.ds(base, BPW)])


def kernel(seq1, seq2, table):
    f = pl.kernel(
        _body,
        out_type=(
            jax.ShapeDtypeStruct((BATCH,), jnp.float32),
            jax.ShapeDtypeStruct((BATCH, EMBED), jnp.float32),
            jax.ShapeDtypeStruct((BATCH, EMBED), jnp.float32),
        ),
        mesh=plsc.VectorSubcoreMesh(core_axis_name="c", subcore_axis_name="s"),
        compiler_params=pltpu.CompilerParams(needs_layout_passes=False,
                                             use_tc_tiling_on_sc=False),
        scratch_types=[
            pltpu.VMEM((NIDX, HPAD), jnp.int32),
            pltpu.VMEM((NROW, HPAD, EMBED), jnp.float32),
            pltpu.VMEM((NSEG, EMBED), jnp.float32),
            pltpu.VMEM((BPW,), jnp.float32),
            pltpu.VMEM((L,), jnp.float32),
            pltpu.SemaphoreType.DMA,
            pltpu.SemaphoreType.DMA,
        ],
    )
    seq12 = jnp.concatenate([seq1.astype(jnp.int32).reshape(-1),
                             seq2.astype(jnp.int32).reshape(-1)])
    return f(seq12, table)
